# Initial kernel scaffold; baseline (speedup 1.0000x reference)
import jax
import jax.numpy as jnp
from jax.experimental import pallas as pl


def kernel(x, edge_index, W1, att_src1, att_dst1, b1, W2, att_src2, att_dst2, b2, Wc, bc):
    raise NotImplementedError("write your pallas kernel here")


# speedup vs baseline: 49.9340x; 49.9340x over previous
import dataclasses
import functools

import jax
import jax.numpy as jnp
from jax import lax
from jax.experimental import pallas as pl
from jax.experimental.pallas import tpu as pltpu
from jax.experimental.pallas import tpu_sc as plsc

N = 10000
NPAD = 10240
E = 320000
NW = 32
BLK = 128
NBLK = 79
EW = BLK * NBLK
EPAD = NW * EW
ROWS_PER_TILE = NPAD // 16


def _tc_embed(x_ref, w_ref, as_ref, ad_ref, h_out, asrc_out, adst_out, m_out):
    h = jnp.dot(x_ref[...], w_ref[...], preferred_element_type=jnp.float32)
    n = h.shape[0]
    asrc = jnp.sum(h * as_ref[...][None, :], axis=1)
    adst = jnp.sum(h * ad_ref[...][None, :], axis=1)
    h_out[pl.ds(0, n), :] = h
    h_out[pl.ds(n, NPAD - n), :] = jnp.zeros((NPAD - n, h.shape[1]), jnp.float32)
    asrc_out[pl.ds(0, n)] = asrc
    asrc_out[pl.ds(n, NPAD - n)] = jnp.zeros((NPAD - n,), jnp.float32)
    adst_out[pl.ds(0, n)] = adst
    adst_out[pl.ds(n, NPAD - n)] = jnp.zeros((NPAD - n,), jnp.float32)
    m = jnp.maximum(jnp.max(asrc) + jnp.max(adst), 0.0)
    m_out[...] = jnp.full((8, 128), m, jnp.float32)


def _tc_layer1(x, W1, att_src1, att_dst1):
    return pl.pallas_call(
        _tc_embed,
        out_shape=(
            jax.ShapeDtypeStruct((NPAD, 32), jnp.float32),
            jax.ShapeDtypeStruct((NPAD,), jnp.float32),
            jax.ShapeDtypeStruct((NPAD,), jnp.float32),
            jax.ShapeDtypeStruct((8, 128), jnp.float32),
        ),
    )(x, W1, att_src1, att_dst1)


def _tc_mid(num_ref, den_ref, h_ref, asrc_ref, adst_ref, m_ref, b_ref,
            w_ref, as_ref, ad_ref, h_out, asrc_out, adst_out, m_out):
    mshift = m_ref[0, 0]
    a_s = asrc_ref[pl.ds(0, N)]
    a_d = adst_ref[pl.ds(0, N)]
    e = a_s + a_d
    w_self = jnp.exp(jnp.maximum(e, 0.2 * e) - mshift)
    h_prev = h_ref[pl.ds(0, N), :]
    num = (num_ref[0, pl.ds(0, N), :] + num_ref[1, pl.ds(0, N), :]
           + w_self[:, None] * h_prev)
    den = den_ref[0, pl.ds(0, N)] + den_ref[1, pl.ds(0, N)] + w_self + 1e-30
    g = jnp.maximum(num / den[:, None] + b_ref[...][None, :], 0.0)
    h2 = jnp.dot(g, w_ref[...], preferred_element_type=jnp.float32)
    asrc = jnp.sum(h2 * as_ref[...][None, :], axis=1)
    adst = jnp.sum(h2 * ad_ref[...][None, :], axis=1)
    h_out[pl.ds(0, N), :] = h2
    h_out[pl.ds(N, NPAD - N), :] = jnp.zeros((NPAD - N, h2.shape[1]), jnp.float32)
    asrc_out[pl.ds(0, N)] = asrc
    asrc_out[pl.ds(N, NPAD - N)] = jnp.zeros((NPAD - N,), jnp.float32)
    adst_out[pl.ds(0, N)] = adst
    adst_out[pl.ds(N, NPAD - N)] = jnp.zeros((NPAD - N,), jnp.float32)
    m = jnp.maximum(jnp.max(asrc) + jnp.max(adst), 0.0)
    m_out[...] = jnp.full((8, 128), m, jnp.float32)


def _tc_layer2(num_p, den_p, h1, asrc1, adst1, m1, b1, W2, att_src2, att_dst2):
    return pl.pallas_call(
        _tc_mid,
        out_shape=(
            jax.ShapeDtypeStruct((NPAD, 32), jnp.float32),
            jax.ShapeDtypeStruct((NPAD,), jnp.float32),
            jax.ShapeDtypeStruct((NPAD,), jnp.float32),
            jax.ShapeDtypeStruct((8, 128), jnp.float32),
        ),
    )(num_p, den_p, h1, asrc1, adst1, m1, b1, W2, att_src2, att_dst2)


def _tc_fin(num_ref, den_ref, h_ref, asrc_ref, adst_ref, m_ref, b_ref,
            wc_ref, bc_ref, out_ref):
    mshift = m_ref[0, 0]
    a_s = asrc_ref[pl.ds(0, N)]
    a_d = adst_ref[pl.ds(0, N)]
    e = a_s + a_d
    w_self = jnp.exp(jnp.maximum(e, 0.2 * e) - mshift)
    h_prev = h_ref[pl.ds(0, N), :]
    num = (num_ref[0, pl.ds(0, N), :] + num_ref[1, pl.ds(0, N), :]
           + w_self[:, None] * h_prev)
    den = den_ref[0, pl.ds(0, N)] + den_ref[1, pl.ds(0, N)] + w_self + 1e-30
    g = jnp.maximum(num / den[:, None] + b_ref[...][None, :], 0.0)
    logits = jnp.dot(g, wc_ref[...], preferred_element_type=jnp.float32)
    out_ref[...] = jax.nn.sigmoid(logits + bc_ref[...][None, :])


def _tc_final(num_p, den_p, h2, asrc2, adst2, m2, b2, Wc, bc):
    return pl.pallas_call(
        _tc_fin,
        out_shape=jax.ShapeDtypeStruct((N, 32), jnp.float32),
    )(num_p, den_p, h2, asrc2, adst2, m2, b2, Wc, bc)


def _sc_body(src_hbm, dst_hbm, asrc_hbm, adst_hbm, h_hbm, m_hbm,
             znum_hbm, zden_hbm, num_out, den_out,
             num_sh, den_sh, asrc_v, adst_v, src_v, dst_v, rows_v, w_v,
             m_v, sem):
    c = lax.axis_index("c")
    s = lax.axis_index("s")
    wid = c * 16 + s

    pltpu.sync_copy(asrc_hbm, asrc_v)
    pltpu.sync_copy(adst_hbm, adst_v)
    pltpu.sync_copy(src_hbm.at[wid], src_v)
    pltpu.sync_copy(dst_hbm.at[wid], dst_v)
    pltpu.sync_copy(m_hbm, m_v)
    pltpu.sync_copy(znum_hbm, num_sh.at[pl.ds(s * ROWS_PER_TILE, ROWS_PER_TILE)])
    pltpu.sync_copy(zden_hbm, den_sh.at[pl.ds(s * ROWS_PER_TILE, ROWS_PER_TILE)])
    plsc.subcore_barrier()

    m16 = m_v[...]

    @pl.loop(0, NBLK)
    def _blk(b):
        gat = pltpu.async_copy(h_hbm.at[src_v.at[b]], rows_v, sem)
        for v in range(BLK // 16):
            sv = src_v[b, pl.ds(v * 16, 16)]
            dv = dst_v[b, pl.ds(v * 16, 16)]
            e = plsc.load_gather(asrc_v, [sv]) + plsc.load_gather(adst_v, [dv])
            e = jnp.maximum(e, 0.2 * e) - m16
            w_v[pl.ds(v * 16, 16)] = jnp.exp(e)
        gat.wait()

        @pl.loop(0, BLK)
        def _scale(j):
            wj = w_v[pl.ds(j, 16)][0]
            rows_v[j, pl.ds(0, 16)] = rows_v[j, pl.ds(0, 16)] * wj
            rows_v[j, pl.ds(16, 16)] = rows_v[j, pl.ds(16, 16)] * wj

        pltpu.sync_copy(rows_v, num_sh.at[dst_v.at[b]], add=True)
        pltpu.sync_copy(w_v.at[pl.ds(0, BLK)], den_sh.at[dst_v.at[b]], add=True)

    plsc.subcore_barrier()
    rows = pl.ds(s * ROWS_PER_TILE, ROWS_PER_TILE)
    pltpu.sync_copy(num_sh.at[rows], num_out.at[c].at[rows])
    pltpu.sync_copy(den_sh.at[rows], den_out.at[c].at[rows])


@functools.partial(jax.jit, static_argnums=())
def _sc_edge_pass(src2d, dst2d, asrc, adst, h, m16, znum, zden):
    mesh = plsc.VectorSubcoreMesh(core_axis_name="c", subcore_axis_name="s")
    cp = pltpu.CompilerParams()
    if "needs_layout_passes" in pltpu.CompilerParams.__dataclass_fields__:
        cp = dataclasses.replace(cp, needs_layout_passes=False)
    if "use_tc_tiling_on_sc" in pltpu.CompilerParams.__dataclass_fields__:
        cp = dataclasses.replace(cp, use_tc_tiling_on_sc=False)
    fn = pl.kernel(
        _sc_body,
        out_type=(
            jax.ShapeDtypeStruct((2, NPAD, 32), jnp.float32),
            jax.ShapeDtypeStruct((2, NPAD), jnp.float32),
        ),
        mesh=mesh,
        scratch_types=[
            pltpu.VMEM_SHARED((NPAD, 32), jnp.float32),
            pltpu.VMEM_SHARED((NPAD,), jnp.float32),
            pltpu.VMEM((NPAD,), jnp.float32),
            pltpu.VMEM((NPAD,), jnp.float32),
            pltpu.VMEM((NBLK, BLK), jnp.int32),
            pltpu.VMEM((NBLK, BLK), jnp.int32),
            pltpu.VMEM((BLK, 32), jnp.float32),
            pltpu.VMEM((BLK + 16,), jnp.float32),
            pltpu.VMEM((16,), jnp.float32),
            pltpu.SemaphoreType.DMA,
        ],
        compiler_params=cp,
    )
    return fn(src2d, dst2d, asrc, adst, h, m16, znum, zden)


def kernel(x, edge_index, W1, att_src1, att_dst1, b1, W2, att_src2, att_dst2,
           b2, Wc, bc):
    src = jnp.asarray(edge_index[0], jnp.int32)
    dst = jnp.asarray(edge_index[1], jnp.int32)
    npadE = EPAD - E
    pad_idx = N + (jnp.arange(npadE, dtype=jnp.int32) % 16)
    src2d = jnp.concatenate([src, pad_idx]).reshape(NW, NBLK, BLK)
    dst2d = jnp.concatenate([dst, pad_idx]).reshape(NW, NBLK, BLK)
    znum = jnp.zeros((ROWS_PER_TILE, 32), jnp.float32)
    zden = jnp.zeros((ROWS_PER_TILE,), jnp.float32)

    h1, asrc1, adst1, m1 = _tc_layer1(x, W1, att_src1, att_dst1)
    num1, den1 = _sc_edge_pass(src2d, dst2d, asrc1, adst1, h1,
                               m1[0, :16], znum, zden)
    h2, asrc2, adst2, m2 = _tc_layer2(num1, den1, h1, asrc1, adst1, m1, b1,
                                      W2, att_src2, att_dst2)
    num2, den2 = _sc_edge_pass(src2d, dst2d, asrc2, adst2, h2,
                               m2[0, :16], znum, zden)
    return _tc_final(num2, den2, h2, asrc2, adst2, m2, b2, Wc, bc)
